# Initial kernel scaffold; baseline (speedup 1.0000x reference)
import jax
import jax.numpy as jnp
from jax.experimental import pallas as pl


def kernel(text_tokens, visual_embeddings, W, b_lin, table):
    raise NotImplementedError("write your pallas kernel here")


# speedup vs baseline: 1.2854x; 1.2854x over previous
import functools

import jax
import jax.numpy as jnp
from jax import lax
from jax.experimental import pallas as pl
from jax.experimental.pallas import tpu as pltpu
from jax.experimental.pallas import tpu_sc as plsc

B = 4
SEQ = 2048
NVIS = 256
HIDDEN = 1024

NC = 2
NS = 16
NW = NC * NS
TOK_PER_W = (B * SEQ) // NW
CHUNK = 32
NCHUNK = TOK_PER_W // CHUNK
VIS_PER_W = (B * NVIS) // NW


def _proj_body(x_ref, w_ref, b_ref, o_ref):
    o_ref[...] = lax.dot_general(
        x_ref[...], w_ref[...], (((1,), (1,)), ((), ())),
        preferred_element_type=jnp.float32,
    ) + b_ref[...]


def _project(visual, w, b_lin):
    x = visual.reshape(B * NVIS, HIDDEN)
    b2 = b_lin.reshape(1, HIDDEN)
    out = pl.pallas_call(
        _proj_body,
        grid=(B,),
        in_specs=[
            pl.BlockSpec((NVIS, HIDDEN), lambda i: (i, 0)),
            pl.BlockSpec((HIDDEN, HIDDEN), lambda i: (0, 0)),
            pl.BlockSpec((1, HIDDEN), lambda i: (0, 0)),
        ],
        out_specs=pl.BlockSpec((NVIS, HIDDEN), lambda i: (i, 0)),
        out_shape=jax.ShapeDtypeStruct((B * NVIS, HIDDEN), jnp.float32),
    )(x, w, b2)
    return out.reshape(B, NVIS, HIDDEN)


_SC_MESH = plsc.VectorSubcoreMesh(core_axis_name="c", subcore_axis_name="s")


@functools.partial(
    pl.kernel,
    mesh=_SC_MESH,
    out_type=jax.ShapeDtypeStruct((B, NVIS + SEQ, HIDDEN), jnp.float32),
    scratch_types=[
        pltpu.VMEM((NCHUNK, CHUNK), jnp.int32),
        pltpu.VMEM((CHUNK, HIDDEN), jnp.float32),
        pltpu.VMEM((CHUNK, HIDDEN), jnp.float32),
        pltpu.VMEM((VIS_PER_W, HIDDEN), jnp.float32),
        pltpu.SemaphoreType.DMA,
    ],
)
def _assemble(tok_hbm, table_hbm, vis_hbm, out_hbm,
              idx_v, buf0, buf1, visbuf, gsem):
    wid = lax.axis_index("s") * NC + lax.axis_index("c")
    b = wid // (NW // B)
    part = wid % (NW // B)

    n0 = part * VIS_PER_W
    pltpu.sync_copy(vis_hbm.at[b, pl.ds(n0, VIS_PER_W), :], visbuf)
    pltpu.sync_copy(visbuf, out_hbm.at[b, pl.ds(n0, VIS_PER_W), :])

    pltpu.sync_copy(tok_hbm.at[wid], idx_v)

    bufs = (buf0, buf1)
    s0 = NVIS + part * TOK_PER_W
    handles = [None] * NCHUNK
    handles[0] = pltpu.async_copy(table_hbm.at[idx_v.at[0]], bufs[0], gsem)
    for j in range(NCHUNK):
        handles[j].wait()
        if j + 1 < NCHUNK:
            handles[j + 1] = pltpu.async_copy(
                table_hbm.at[idx_v.at[j + 1]], bufs[(j + 1) % 2], gsem)
        pltpu.sync_copy(bufs[j % 2],
                        out_hbm.at[b, pl.ds(s0 + j * CHUNK, CHUNK), :])


def kernel(text_tokens, visual_embeddings, W, b_lin, table):
    projected = _project(visual_embeddings, W, b_lin)
    tok = text_tokens.astype(jnp.int32).reshape(NW, NCHUNK, CHUNK)
    return _assemble(tok, table, projected)


# speedup vs baseline: 1.3582x; 1.0566x over previous
import functools

import jax
import jax.numpy as jnp
from jax import lax
from jax.experimental import pallas as pl
from jax.experimental.pallas import tpu as pltpu
from jax.experimental.pallas import tpu_sc as plsc

B = 4
SEQ = 2048
NVIS = 256
HIDDEN = 1024

NC = 2
NS = 16
NW = NC * NS
TOK_PER_W = (B * SEQ) // NW
CHUNK = 32
NCHUNK = TOK_PER_W // CHUNK


_SC_MESH = plsc.VectorSubcoreMesh(core_axis_name="c", subcore_axis_name="s")


@functools.partial(
    pl.kernel,
    mesh=_SC_MESH,
    out_type=jax.ShapeDtypeStruct((B, NVIS + SEQ, HIDDEN), jnp.float32),
    scratch_types=[
        pltpu.VMEM((NCHUNK, CHUNK), jnp.int32),
        pltpu.VMEM((CHUNK, HIDDEN), jnp.float32),
        pltpu.VMEM((CHUNK, HIDDEN), jnp.float32),
        pltpu.SemaphoreType.DMA,
    ],
)
def _gather_txt(tok_hbm, table_hbm, out_hbm, idx_v, buf0, buf1, gsem):
    wid = lax.axis_index("s") * NC + lax.axis_index("c")
    b = wid // (NW // B)
    part = wid % (NW // B)

    pltpu.sync_copy(tok_hbm.at[wid], idx_v)

    bufs = (buf0, buf1)
    s0 = NVIS + part * TOK_PER_W
    handles = [None] * NCHUNK
    handles[0] = pltpu.async_copy(table_hbm.at[idx_v.at[0]], bufs[0], gsem)
    for j in range(NCHUNK):
        handles[j].wait()
        if j + 1 < NCHUNK:
            handles[j + 1] = pltpu.async_copy(
                table_hbm.at[idx_v.at[j + 1]], bufs[(j + 1) % 2], gsem)
        pltpu.sync_copy(bufs[j % 2],
                        out_hbm.at[b, pl.ds(s0 + j * CHUNK, CHUNK), :])


def _proj_body(big_ref, x_ref, w_ref, b_ref, o_ref):
    del big_ref
    o_ref[0] = lax.dot_general(
        x_ref[0], w_ref[...], (((1,), (1,)), ((), ())),
        preferred_element_type=jnp.float32,
    ) + b_ref[...]


def _project_into(big, visual, w, b_lin):
    b2 = b_lin.reshape(1, HIDDEN)
    return pl.pallas_call(
        _proj_body,
        grid=(B,),
        in_specs=[
            pl.BlockSpec(memory_space=pl.ANY),
            pl.BlockSpec((1, NVIS, HIDDEN), lambda i: (i, 0, 0)),
            pl.BlockSpec((HIDDEN, HIDDEN), lambda i: (0, 0)),
            pl.BlockSpec((1, HIDDEN), lambda i: (0, 0)),
        ],
        out_specs=pl.BlockSpec((1, NVIS, HIDDEN), lambda i: (i, 0, 0)),
        out_shape=jax.ShapeDtypeStruct((B, NVIS + SEQ, HIDDEN), jnp.float32),
        input_output_aliases={0: 0},
    )(big, visual, w, b2)


def kernel(text_tokens, visual_embeddings, W, b_lin, table):
    tok = text_tokens.astype(jnp.int32).reshape(NW, NCHUNK, CHUNK)
    big = _gather_txt(tok, table)
    return _project_into(big, visual_embeddings, W, b_lin)


# speedup vs baseline: 1.4512x; 1.0685x over previous
import functools

import jax
import jax.numpy as jnp
from jax import lax
from jax.experimental import pallas as pl
from jax.experimental.pallas import tpu as pltpu
from jax.experimental.pallas import tpu_sc as plsc

B = 4
SEQ = 2048
NVIS = 256
HIDDEN = 1024

NC = 2
NS = 16
NW = NC * NS
TOK_PER_W = (B * SEQ) // NW
CHUNK = 32
NCHUNK = TOK_PER_W // CHUNK


_SC_MESH = plsc.VectorSubcoreMesh(core_axis_name="c", subcore_axis_name="s")


@functools.partial(
    pl.kernel,
    mesh=_SC_MESH,
    out_type=jax.ShapeDtypeStruct((B, NVIS + SEQ, HIDDEN), jnp.float32),
    scratch_types=[
        pltpu.VMEM((NCHUNK, CHUNK), jnp.int32),
        pltpu.VMEM((CHUNK, HIDDEN), jnp.float32),
        pltpu.VMEM((CHUNK, HIDDEN), jnp.float32),
        pltpu.SemaphoreType.DMA,
    ],
)
def _gather_txt(tok_hbm, table_hbm, out_hbm, idx_v, buf0, buf1, gsem):
    wid = lax.axis_index("s") * NC + lax.axis_index("c")
    b = wid // (NW // B)
    part = wid % (NW // B)

    pltpu.sync_copy(tok_hbm.at[wid], idx_v)

    bufs = (buf0, buf1)
    s0 = NVIS + part * TOK_PER_W
    handles = [None] * NCHUNK
    handles[0] = pltpu.async_copy(table_hbm.at[idx_v.at[0]], bufs[0], gsem)
    for j in range(NCHUNK):
        handles[j].wait()
        if j + 1 < NCHUNK:
            handles[j + 1] = pltpu.async_copy(
                table_hbm.at[idx_v.at[j + 1]], bufs[(j + 1) % 2], gsem)
        pltpu.sync_copy(bufs[j % 2],
                        out_hbm.at[b, pl.ds(s0 + j * CHUNK, CHUNK), :])


def _proj_body(x_ref, w_ref, b_ref, o_ref):
    o_ref[0] = lax.dot_general(
        x_ref[0], w_ref[...], (((1,), (1,)), ((), ())),
        preferred_element_type=jnp.float32,
    ) + b_ref[...]


def _project(visual, w, b_lin):
    b2 = b_lin.reshape(1, HIDDEN)
    return pl.pallas_call(
        _proj_body,
        grid=(B,),
        in_specs=[
            pl.BlockSpec((1, NVIS, HIDDEN), lambda i: (i, 0, 0)),
            pl.BlockSpec((HIDDEN, HIDDEN), lambda i: (0, 0)),
            pl.BlockSpec((1, HIDDEN), lambda i: (0, 0)),
        ],
        out_specs=pl.BlockSpec((1, NVIS, HIDDEN), lambda i: (i, 0, 0)),
        out_shape=jax.ShapeDtypeStruct((B, NVIS, HIDDEN), jnp.float32),
    )(visual, w, b2)


def _stitch_body(big_ref, vis_ref, o_ref):
    del big_ref
    o_ref[...] = vis_ref[...]


def _stitch(big, vis):
    return pl.pallas_call(
        _stitch_body,
        grid=(1,),
        in_specs=[
            pl.BlockSpec(memory_space=pl.ANY),
            pl.BlockSpec((B, NVIS, HIDDEN), lambda i: (0, 0, 0)),
        ],
        out_specs=pl.BlockSpec((B, NVIS, HIDDEN), lambda i: (0, 0, 0)),
        out_shape=jax.ShapeDtypeStruct((B, NVIS + SEQ, HIDDEN), jnp.float32),
        input_output_aliases={0: 0},
    )(big, vis)


def kernel(text_tokens, visual_embeddings, W, b_lin, table):
    tok = text_tokens.astype(jnp.int32).reshape(NW, NCHUNK, CHUNK)
    big = _gather_txt(tok, table)
    vis = _project(visual_embeddings, W, b_lin)
    return _stitch(big, vis)


# speedup vs baseline: 1.4874x; 1.0250x over previous
import functools

import jax
import jax.numpy as jnp
from jax import lax
from jax.experimental import pallas as pl
from jax.experimental.pallas import tpu as pltpu
from jax.experimental.pallas import tpu_sc as plsc

B = 4
SEQ = 2048
NVIS = 256
HIDDEN = 1024

NC = 2
NS = 16
NW = NC * NS
TOK_PER_W = (B * SEQ) // NW
CHUNKS = (56, 56, 56, 56, 32)
BUF_ROWS = max(CHUNKS)
NCHUNK = len(CHUNKS)


_SC_MESH = plsc.VectorSubcoreMesh(core_axis_name="c", subcore_axis_name="s")


@functools.partial(
    pl.kernel,
    mesh=_SC_MESH,
    out_type=jax.ShapeDtypeStruct((B, NVIS + SEQ, HIDDEN), jnp.float32),
    scratch_types=[
        pltpu.VMEM((TOK_PER_W,), jnp.int32),
        pltpu.VMEM((BUF_ROWS, HIDDEN), jnp.float32),
        pltpu.VMEM((BUF_ROWS, HIDDEN), jnp.float32),
        pltpu.SemaphoreType.DMA,
    ],
)
def _gather_txt(tok_hbm, table_hbm, out_hbm, idx_v, buf0, buf1, gsem):
    wid = lax.axis_index("s") * NC + lax.axis_index("c")
    b = wid // (NW // B)
    part = wid % (NW // B)

    pltpu.sync_copy(tok_hbm.at[wid], idx_v)

    bufs = (buf0, buf1)
    offs = [sum(CHUNKS[:j]) for j in range(NCHUNK)]
    s0 = NVIS + part * TOK_PER_W

    def _gather(j):
        ch = CHUNKS[j]
        dst = bufs[j % 2] if ch == BUF_ROWS else bufs[j % 2].at[pl.ds(0, ch), :]
        return pltpu.async_copy(
            table_hbm.at[idx_v.at[pl.ds(offs[j], ch)]], dst, gsem)

    handles = [None] * NCHUNK
    handles[0] = _gather(0)
    for j in range(NCHUNK):
        ch = CHUNKS[j]
        handles[j].wait()
        if j + 1 < NCHUNK:
            handles[j + 1] = _gather(j + 1)
        src = bufs[j % 2] if ch == BUF_ROWS else bufs[j % 2].at[pl.ds(0, ch), :]
        pltpu.sync_copy(src, out_hbm.at[b, pl.ds(s0 + offs[j], ch), :])


def _proj_body(x_ref, w_ref, b_ref, o_ref):
    o_ref[0] = lax.dot_general(
        x_ref[0], w_ref[...], (((1,), (1,)), ((), ())),
        preferred_element_type=jnp.float32,
    ) + b_ref[...]


def _project(visual, w, b_lin):
    b2 = b_lin.reshape(1, HIDDEN)
    return pl.pallas_call(
        _proj_body,
        grid=(B,),
        in_specs=[
            pl.BlockSpec((1, NVIS, HIDDEN), lambda i: (i, 0, 0)),
            pl.BlockSpec((HIDDEN, HIDDEN), lambda i: (0, 0)),
            pl.BlockSpec((1, HIDDEN), lambda i: (0, 0)),
        ],
        out_specs=pl.BlockSpec((1, NVIS, HIDDEN), lambda i: (i, 0, 0)),
        out_shape=jax.ShapeDtypeStruct((B, NVIS, HIDDEN), jnp.float32),
    )(visual, w, b2)


def _stitch_body(big_ref, vis_ref, o_ref):
    del big_ref
    o_ref[...] = vis_ref[...]


def _stitch(big, vis):
    return pl.pallas_call(
        _stitch_body,
        grid=(1,),
        in_specs=[
            pl.BlockSpec(memory_space=pl.ANY),
            pl.BlockSpec((B, NVIS, HIDDEN), lambda i: (0, 0, 0)),
        ],
        out_specs=pl.BlockSpec((B, NVIS, HIDDEN), lambda i: (0, 0, 0)),
        out_shape=jax.ShapeDtypeStruct((B, NVIS + SEQ, HIDDEN), jnp.float32),
        input_output_aliases={0: 0},
    )(big, vis)


def kernel(text_tokens, visual_embeddings, W, b_lin, table):
    tok = text_tokens.astype(jnp.int32).reshape(NW, TOK_PER_W)
    big = _gather_txt(tok, table)
    vis = _project(visual_embeddings, W, b_lin)
    return _stitch(big, vis)


# speedup vs baseline: 1.4945x; 1.0047x over previous
import functools

import jax
import jax.numpy as jnp
from jax import lax
from jax.experimental import pallas as pl
from jax.experimental.pallas import tpu as pltpu
from jax.experimental.pallas import tpu_sc as plsc

B = 4
SEQ = 2048
NVIS = 256
HIDDEN = 1024

NC = 2
NS = 16
NW = NC * NS
TOK_PER_W = (B * SEQ) // NW
CHUNKS = (56, 56, 56, 56, 32)
BUF_ROWS = max(CHUNKS)
NCHUNK = len(CHUNKS)


_SC_MESH = plsc.VectorSubcoreMesh(core_axis_name="c", subcore_axis_name="s")


@functools.partial(
    pl.kernel,
    mesh=_SC_MESH,
    out_type=jax.ShapeDtypeStruct((B, NVIS + SEQ, HIDDEN), jnp.float32),
    scratch_types=[
        pltpu.VMEM((TOK_PER_W,), jnp.int32),
        pltpu.VMEM((BUF_ROWS, HIDDEN), jnp.float32),
        pltpu.VMEM((BUF_ROWS, HIDDEN), jnp.float32),
        pltpu.SemaphoreType.DMA,
    ],
)
def _gather_txt(tok_hbm, table_hbm, out_hbm, idx_v, buf0, buf1, gsem):
    wid = lax.axis_index("s") * NC + lax.axis_index("c")
    b = wid // (NW // B)
    part = wid % (NW // B)

    pltpu.sync_copy(tok_hbm.at[b, pl.ds(part * TOK_PER_W, TOK_PER_W)], idx_v)

    bufs = (buf0, buf1)
    offs = [sum(CHUNKS[:j]) for j in range(NCHUNK)]
    s0 = NVIS + part * TOK_PER_W

    def _gather(j):
        ch = CHUNKS[j]
        dst = bufs[j % 2] if ch == BUF_ROWS else bufs[j % 2].at[pl.ds(0, ch), :]
        return pltpu.async_copy(
            table_hbm.at[idx_v.at[pl.ds(offs[j], ch)]], dst, gsem)

    handles = [None] * NCHUNK
    handles[0] = _gather(0)
    for j in range(NCHUNK):
        ch = CHUNKS[j]
        handles[j].wait()
        if j + 1 < NCHUNK:
            handles[j + 1] = _gather(j + 1)
        src = bufs[j % 2] if ch == BUF_ROWS else bufs[j % 2].at[pl.ds(0, ch), :]
        pltpu.sync_copy(src, out_hbm.at[b, pl.ds(s0 + offs[j], ch), :])


def _proj_body(x_ref, w_ref, b_ref, o_ref):
    o_ref[0] = lax.dot_general(
        x_ref[0], w_ref[...], (((1,), (1,)), ((), ())),
        preferred_element_type=jnp.float32,
    ) + b_ref[...]


def _project(visual, w, b_lin):
    return pl.pallas_call(
        _proj_body,
        grid=(B,),
        in_specs=[
            pl.BlockSpec((1, NVIS, HIDDEN), lambda i: (i, 0, 0)),
            pl.BlockSpec((HIDDEN, HIDDEN), lambda i: (0, 0)),
            pl.BlockSpec((HIDDEN,), lambda i: (0,)),
        ],
        out_specs=pl.BlockSpec((1, NVIS, HIDDEN), lambda i: (i, 0, 0)),
        out_shape=jax.ShapeDtypeStruct((B, NVIS, HIDDEN), jnp.float32),
    )(visual, w, b_lin)


def _stitch_body(big_ref, vis_ref, o_ref):
    del big_ref
    o_ref[...] = vis_ref[...]


def _stitch(big, vis):
    return pl.pallas_call(
        _stitch_body,
        grid=(1,),
        in_specs=[
            pl.BlockSpec(memory_space=pl.ANY),
            pl.BlockSpec((B, NVIS, HIDDEN), lambda i: (0, 0, 0)),
        ],
        out_specs=pl.BlockSpec((B, NVIS, HIDDEN), lambda i: (0, 0, 0)),
        out_shape=jax.ShapeDtypeStruct((B, NVIS + SEQ, HIDDEN), jnp.float32),
        input_output_aliases={0: 0},
    )(big, vis)


def kernel(text_tokens, visual_embeddings, W, b_lin, table):
    big = _gather_txt(text_tokens, table)
    vis = _project(visual_embeddings, W, b_lin)
    return _stitch(big, vis)
